# baseline (speedup 1.0000x reference)
import jax
import jax.numpy as jnp
from jax import lax
from jax.experimental import pallas as pl
from jax.experimental.pallas import tpu as pltpu
from jax.experimental.pallas import tpu_sc as plsc

_BINS = 256
_B = 8
_N = 32768
_LANES = 16
_NS = 16
_TILES_PER_ROW = 2
_CHUNK = _N // _TILES_PER_ROW
_SD = 390.625
_PBINS = _BINS + 2


def _body(x_hbm, m_hbm, out_hbm, x_v, m_v, acc_v, part_v, red_v, out_v,
          shared, sem1, sem2):
    c = lax.axis_index("c")
    s = lax.axis_index("s")
    row = c * 8 + s // _TILES_PER_ROW
    off = (s % _TILES_PER_ROW) * _CHUNK

    cp1 = pltpu.async_copy(x_hbm.at[row, pl.ds(off, _CHUNK)], x_v, sem1)
    cp2 = pltpu.async_copy(m_hbm.at[row, pl.ds(off, _CHUNK)], m_v, sem2)

    zero16 = jnp.zeros((_LANES,), jnp.float32)

    @plsc.parallel_loop(0, _LANES * _PBINS, step=_LANES, unroll=4)
    def _zero(k):
        acc_v[pl.ds(k, _LANES)] = zero16

    cp1.wait()
    cp2.wait()

    lane_base = lax.iota(jnp.int32, _LANES) * _PBINS + 1
    one = jnp.float32(1.0)
    sd = jnp.float32(_SD)
    magic = jnp.float32(8388608.0)

    @plsc.parallel_loop(0, _CHUNK, step=_LANES, unroll=4)
    def _step(k):
        xv = x_v[pl.ds(k, _LANES)]
        mv = m_v[pl.ds(k, _LANES)]
        t = xv * jnp.float32(_BINS)
        fe = (t + magic) - magic
        e = fe.astype(jnp.int32)
        r = t - fe
        u = r * (-sd)
        q = jnp.exp(jnp.minimum(u, -u))
        w_far = q / (one + q)
        w_e = jnp.where(r >= 0, one - w_far, w_far)
        we_m = w_e * mv
        idx = lane_base + e
        plsc.addupdate_scatter(acc_v, [idx], we_m)
        plsc.addupdate_scatter(acc_v, [idx - 1], mv - we_m)

    @plsc.parallel_loop(0, _BINS, step=_LANES, unroll=2)
    def _red(col):
        v = acc_v[pl.ds(col + 1, _LANES)]
        for l in range(1, _LANES):
            v = v + acc_v[pl.ds(l * _PBINS + col + 1, _LANES)]
        part_v[pl.ds(col, _LANES)] = v

    pltpu.sync_copy(part_v, shared.at[s])
    plsc.subcore_barrier()

    @pl.when(s < 8)
    def _():
        pltpu.sync_copy(shared.at[pl.ds(s * _TILES_PER_ROW, _TILES_PER_ROW)],
                        red_v)

        @plsc.parallel_loop(0, _BINS, step=_LANES, unroll=2)
        def _out(col):
            v = red_v[0, pl.ds(col, _LANES)]
            for l in range(1, _TILES_PER_ROW):
                v = v + red_v[l, pl.ds(col, _LANES)]
            out_v[pl.ds(col, _LANES)] = v

        pltpu.sync_copy(out_v, out_hbm.at[c * 8 + s])


@jax.jit
def _soft_hist(x, mask):
    mesh = plsc.VectorSubcoreMesh(core_axis_name="c", subcore_axis_name="s",
                              num_cores=1)
    f = pl.kernel(
        _body,
        out_type=jax.ShapeDtypeStruct((_B, _BINS), jnp.float32),
        mesh=mesh,
        compiler_params=pltpu.CompilerParams(
            needs_layout_passes=False, skip_device_barrier=True),
        scratch_types=[
            pltpu.VMEM((_CHUNK,), jnp.float32),
            pltpu.VMEM((_CHUNK,), jnp.float32),
            pltpu.VMEM((_LANES * _PBINS,), jnp.float32),
            pltpu.VMEM((_BINS,), jnp.float32),
            pltpu.VMEM((_TILES_PER_ROW, _BINS), jnp.float32),
            pltpu.VMEM((_BINS,), jnp.float32),
            pltpu.VMEM_SHARED((_NS, _BINS), jnp.float32),
            pltpu.SemaphoreType.DMA,
            pltpu.SemaphoreType.DMA,
        ],
    )
    return f(x, mask)


def kernel(x, mask):
    return _soft_hist(x, mask)


# speedup vs baseline: 1.0697x; 1.0697x over previous
import jax
import jax.numpy as jnp
from jax import lax
from jax.experimental import pallas as pl
from jax.experimental.pallas import tpu as pltpu
from jax.experimental.pallas import tpu_sc as plsc

_BINS = 256
_B = 8
_N = 32768
_LANES = 16
_NS = 16
_TILES_PER_ROW = 4
_CHUNK = _N // _TILES_PER_ROW
_SD = 390.625
_PBINS = _BINS + 2


def _body(x_hbm, m_hbm, out_hbm, x_v, m_v, acc_v, part_v, red_v, out_v,
          shared, sem1, sem2):
    c = lax.axis_index("c")
    s = lax.axis_index("s")
    row = c * 4 + s // _TILES_PER_ROW
    off = (s % _TILES_PER_ROW) * _CHUNK

    cp1 = pltpu.async_copy(x_hbm.at[row, pl.ds(off, _CHUNK)], x_v, sem1)
    cp2 = pltpu.async_copy(m_hbm.at[row, pl.ds(off, _CHUNK)], m_v, sem2)

    zero16 = jnp.zeros((_LANES,), jnp.float32)

    @plsc.parallel_loop(0, _LANES * _PBINS, step=_LANES, unroll=4)
    def _zero(k):
        acc_v[pl.ds(k, _LANES)] = zero16

    cp1.wait()
    cp2.wait()

    lane_base = lax.iota(jnp.int32, _LANES) * _PBINS + 1
    one = jnp.float32(1.0)
    sd = jnp.float32(_SD)
    magic = jnp.float32(8388608.0)

    @plsc.parallel_loop(0, _CHUNK, step=_LANES, unroll=4)
    def _step(k):
        xv = x_v[pl.ds(k, _LANES)]
        mv = m_v[pl.ds(k, _LANES)]
        t = xv * jnp.float32(_BINS)
        fe = (t + magic) - magic
        e = fe.astype(jnp.int32)
        r = t - fe
        u = r * (-sd)
        q = jnp.exp(jnp.minimum(u, -u))
        w_far = q / (one + q)
        w_e = jnp.where(r >= 0, one - w_far, w_far)
        we_m = w_e * mv
        idx = lane_base + e
        plsc.addupdate_scatter(acc_v, [idx], we_m)
        plsc.addupdate_scatter(acc_v, [idx - 1], mv - we_m)

    @plsc.parallel_loop(0, _BINS, step=_LANES, unroll=2)
    def _red(col):
        v = acc_v[pl.ds(col + 1, _LANES)]
        for l in range(1, _LANES):
            v = v + acc_v[pl.ds(l * _PBINS + col + 1, _LANES)]
        part_v[pl.ds(col, _LANES)] = v

    pltpu.sync_copy(part_v, shared.at[s])
    plsc.subcore_barrier()

    @pl.when(s < _TILES_PER_ROW)
    def _():
        pltpu.sync_copy(shared.at[pl.ds(s * _TILES_PER_ROW, _TILES_PER_ROW)],
                        red_v)

        @plsc.parallel_loop(0, _BINS, step=_LANES, unroll=2)
        def _out(col):
            v = red_v[0, pl.ds(col, _LANES)]
            for l in range(1, _TILES_PER_ROW):
                v = v + red_v[l, pl.ds(col, _LANES)]
            out_v[pl.ds(col, _LANES)] = v

        pltpu.sync_copy(out_v, out_hbm.at[c * 4 + s])


@jax.jit
def _soft_hist(x, mask):
    mesh = plsc.VectorSubcoreMesh(core_axis_name="c", subcore_axis_name="s")
    f = pl.kernel(
        _body,
        out_type=jax.ShapeDtypeStruct((_B, _BINS), jnp.float32),
        mesh=mesh,
        compiler_params=pltpu.CompilerParams(
            needs_layout_passes=False, skip_device_barrier=True),
        scratch_types=[
            pltpu.VMEM((_CHUNK,), jnp.float32),
            pltpu.VMEM((_CHUNK,), jnp.float32),
            pltpu.VMEM((_LANES * _PBINS,), jnp.float32),
            pltpu.VMEM((_BINS,), jnp.float32),
            pltpu.VMEM((_TILES_PER_ROW, _BINS), jnp.float32),
            pltpu.VMEM((_BINS,), jnp.float32),
            pltpu.VMEM_SHARED((_NS, _BINS), jnp.float32),
            pltpu.SemaphoreType.DMA,
            pltpu.SemaphoreType.DMA,
        ],
    )
    return f(x, mask)


def kernel(x, mask):
    return _soft_hist(x, mask)
